# baseline (speedup 1.0000x reference)
import functools

import jax
import jax.numpy as jnp
from jax import lax
from jax.experimental import pallas as pl
import jax.experimental.pallas.tpu as pltpu

S_BLK = 256
D = 4096
W_LANES = 128
NBUF = 5
B = 2
S = 4096
NCH_PER_B = S // S_BLK
NCH = B * NCH_PER_B


def _scan_chunk(in_buf, out_buf, slot, carry):
    carries = []
    for c in range(D // W_LANES):
        sl = pl.ds(c * W_LANES, W_LANES)
        acc = in_buf[slot, :, sl]
        k = 1
        while k < S_BLK:
            shifted = jnp.pad(acc, ((k, 0), (0, 0)))[:S_BLK]
            acc = acc + shifted
            k *= 2
        cc = carry[:, c * W_LANES : (c + 1) * W_LANES]
        out_buf[slot, :, sl] = acc + cc
        carries.append(cc + acc[S_BLK - 1 :, :])
    return jnp.concatenate(carries, axis=1)


def _pipeline_body(x_hbm, o_hbm, in_buf, out_buf, in_sems, out_sems):
    def in_copy(c):
        b = c // NCH_PER_B
        s0 = (c % NCH_PER_B) * S_BLK
        slot = c % NBUF
        return pltpu.make_async_copy(
            x_hbm.at[b, pl.ds(s0, S_BLK), :],
            in_buf.at[slot],
            in_sems.at[slot],
        )

    def out_copy(c):
        b = c // NCH_PER_B
        s0 = (c % NCH_PER_B) * S_BLK
        slot = c % NBUF
        return pltpu.make_async_copy(
            out_buf.at[slot],
            o_hbm.at[b, pl.ds(s0, S_BLK), :],
            out_sems.at[slot],
        )

    in_copy(0).start()
    in_copy(1).start()

    carry = jnp.zeros((1, D), jnp.float32)
    for c in range(NCH):
        slot = c % NBUF
        if c % NCH_PER_B == 0:
            carry = jnp.zeros((1, D), jnp.float32)
        in_copy(c).wait()
        if c + 2 < NCH:
            in_copy(c + 2).start()
        if c >= NBUF:
            out_copy(c - NBUF).wait()
        carry = _scan_chunk(in_buf, out_buf, slot, carry)
        out_copy(c).start()

    for c in range(NCH - NBUF, NCH):
        out_copy(c).wait()


@jax.jit
def kernel(x):
    return pl.pallas_call(
        _pipeline_body,
        in_specs=[pl.BlockSpec(memory_space=pl.ANY)],
        out_specs=pl.BlockSpec(memory_space=pl.ANY),
        out_shape=jax.ShapeDtypeStruct(x.shape, x.dtype),
        scratch_shapes=[
            pltpu.VMEM((NBUF, S_BLK, D), jnp.float32),
            pltpu.VMEM((NBUF, S_BLK, D), jnp.float32),
            pltpu.SemaphoreType.DMA((NBUF,)),
            pltpu.SemaphoreType.DMA((NBUF,)),
        ],
    )(x)
